# baseline (speedup 1.0000x reference)
import functools

import jax
import jax.numpy as jnp
from jax.experimental import pallas as pl
from jax.experimental.pallas import tpu as pltpu

_VMEM_LIMIT = 32 * 2 ** 20


def _ceil_to(x, m):
    return (x + m - 1) // m * m


def _plan_rows(m):
    if m <= 512:
        return m, m
    for t in (512, 256, 224, 128, 112, 64, 32, 16):
        if m % t == 0:
            return t, m
    return 512, _ceil_to(m, 512)


def _plan_cols(n):
    assert n % 128 == 0, n
    return (256 if n % 256 == 0 and n > 128 else 128), n


def _mm_body(relu, has_res, *refs):
    if has_res:
        a_ref, w_ref, b_ref, r_ref, o_ref = refs
    else:
        a_ref, w_ref, b_ref, o_ref = refs
        r_ref = None
    acc = jnp.dot(a_ref[...], w_ref[...], preferred_element_type=jnp.float32)
    acc = acc + b_ref[...]
    if r_ref is not None:
        acc = acc + r_ref[...].astype(jnp.float32)
    if relu:
        acc = jnp.maximum(acc, 0.0)
    o_ref[...] = acc.astype(o_ref.dtype)


def _fused_mm(a, w, bias, residual=None, relu=False, out_dtype=jnp.bfloat16):
    M, K = a.shape
    Kw, N = w.shape
    assert K == Kw, (a.shape, w.shape)
    tm, Mp = _plan_rows(M)
    tn, Np = _plan_cols(N)

    a = a.astype(jnp.bfloat16)
    if Mp != M:
        a = jnp.pad(a, ((0, Mp - M), (0, 0)))
    b2 = bias.reshape(1, N).astype(jnp.float32)

    ins = [a, w.astype(jnp.bfloat16), b2]
    specs = [pl.BlockSpec((tm, K), lambda i, j: (i, 0)),
             pl.BlockSpec((K, tn), lambda i, j: (0, j)),
             pl.BlockSpec((1, tn), lambda i, j: (0, j))]
    has_res = residual is not None
    if has_res:
        r = residual
        if Mp != M:
            r = jnp.pad(r, ((0, Mp - M), (0, 0)))
        ins.append(r)
        specs.append(pl.BlockSpec((tm, tn), lambda i, j: (i, j)))

    out = pl.pallas_call(
        functools.partial(_mm_body, relu, has_res),
        out_shape=jax.ShapeDtypeStruct((Mp, Np), out_dtype),
        grid=(Mp // tm, Np // tn),
        in_specs=specs,
        out_specs=pl.BlockSpec((tm, tn), lambda i, j: (i, j)),
        compiler_params=pltpu.CompilerParams(
            dimension_semantics=("parallel", "parallel"),
            vmem_limit_bytes=_VMEM_LIMIT),
    )(*ins)
    if Mp != M:
        out = out[:M]
    return out


def _c3_body(wp, x_ref, w_ref, b_ref, o_ref):
    m_f = o_ref.shape[1]
    acc = None
    for di in range(3):
        for dj in range(3):
            off = di * wp + dj
            part = jnp.dot(x_ref[0, off:off + m_f, :], w_ref[di * 3 + dj],
                           preferred_element_type=jnp.float32)
            acc = part if acc is None else acc + part
    acc = jnp.maximum(acc + b_ref[...], 0.0)
    o_ref[...] = acc[None].astype(o_ref.dtype)


def _conv3x3_s1(x, w, bias):
    N, H, W, C = x.shape
    Wp = W + 2
    xp = jnp.pad(x, ((0, 0), (1, 1), (1, 1), (0, 0)))
    xf = xp.reshape(N, (H + 2) * Wp, C)
    xf = jnp.pad(xf, ((0, 0), (0, Wp), (0, 0)))
    L = (H + 3) * Wp
    m_f = H * Wp

    cout = w.shape[-1]
    tn, _ = _plan_cols(cout)
    out = pl.pallas_call(
        functools.partial(_c3_body, Wp),
        out_shape=jax.ShapeDtypeStruct((N, m_f, cout), jnp.bfloat16),
        grid=(N, cout // tn),
        in_specs=[pl.BlockSpec((1, L, C), lambda n, j: (n, 0, 0)),
                  pl.BlockSpec((9, C, tn), lambda n, j: (0, 0, j)),
                  pl.BlockSpec((1, tn), lambda n, j: (0, j))],
        out_specs=pl.BlockSpec((1, m_f, tn), lambda n, j: (n, 0, j)),
        compiler_params=pltpu.CompilerParams(
            dimension_semantics=("parallel", "parallel"),
            vmem_limit_bytes=_VMEM_LIMIT),
    )(xf, w, bias.reshape(1, cout))
    return out.reshape(N, H, Wp, cout)[:, :, :W, :]


def _conv1x1(x, w, bias, stride=1, relu=True, residual=None):
    xs = x if stride == 1 else x[:, ::stride, ::stride, :]
    N, Ho, Wo, C = xs.shape
    a = xs.reshape(N * Ho * Wo, C)
    kp, cout = w.shape
    if kp != C:
        a = jnp.pad(a, ((0, 0), (0, kp - C)))
    r2 = residual.reshape(N * Ho * Wo, cout) if residual is not None else None
    y = _fused_mm(a, w, bias, residual=r2, relu=relu)
    return y.reshape(N, Ho, Wo, cout)


def _conv_im2col(x, w, bias, kh, kw, stride, pad, relu=True):
    N, H, W, C = x.shape
    xp = jnp.pad(x, ((0, 0), (pad, pad), (pad, pad), (0, 0)))
    Ho = (H + 2 * pad - kh) // stride + 1
    Wo = (W + 2 * pad - kw) // stride + 1
    cols = [xp[:, i:i + (Ho - 1) * stride + 1:stride,
               j:j + (Wo - 1) * stride + 1:stride, :]
            for i in range(kh) for j in range(kw)]
    kp = w.shape[0]
    if kp != kh * kw * C:
        cols.append(jnp.zeros((N, Ho, Wo, kp - kh * kw * C), x.dtype))
    a = jnp.concatenate(cols, axis=-1).reshape(N * Ho * Wo, kp)
    y = _fused_mm(a, w, bias, relu=relu)
    return y.reshape(N, Ho, Wo, w.shape[1])


def _maxpool_3x3_s2(x):
    N, H, W, C = x.shape
    xp = jnp.pad(x, ((0, 0), (1, 1), (1, 1), (0, 0)),
                 constant_values=-jnp.inf)
    Ho = (H + 2 - 3) // 2 + 1
    Wo = (W + 2 - 3) // 2 + 1
    out = None
    for i in range(3):
        for j in range(3):
            win = xp[:, i:i + 2 * Ho - 1:2, j:j + 2 * Wo - 1:2, :]
            out = win if out is None else jnp.maximum(out, win)
    return out


def _head_body(inv_hw, x_ref, w_ref, b_ref, o_ref):
    feat = jnp.sum(x_ref[...].astype(jnp.float32), axis=1) * inv_hw
    o_ref[...] = jnp.dot(feat.astype(jnp.bfloat16), w_ref[...],
                         preferred_element_type=jnp.float32) + b_ref[...]


def _avgpool_linear(x, w, bias):
    N, H, W, C = x.shape
    hw = H * W
    E = w.shape[1]
    return pl.pallas_call(
        functools.partial(_head_body, 1.0 / hw),
        out_shape=jax.ShapeDtypeStruct((N, E), jnp.float32),
        grid=(1,),
        in_specs=[pl.BlockSpec((N, hw, C), lambda i: (0, 0, 0)),
                  pl.BlockSpec((C, E), lambda i: (0, 0)),
                  pl.BlockSpec((1, E), lambda i: (0, 0))],
        out_specs=pl.BlockSpec((N, E), lambda i: (0, 0)),
        compiler_params=pltpu.CompilerParams(
            dimension_semantics=("arbitrary",),
            vmem_limit_bytes=_VMEM_LIMIT),
    )(x.reshape(N, hw, C), w.astype(jnp.bfloat16),
      bias.reshape(1, E).astype(jnp.float32))


_LAYER_CFG = [(3, 1), (4, 2), (6, 2), (3, 2)]


def kernel(images, conv1_w, conv1_b, l0_b0_c1_w, l0_b0_c1_b, l0_b0_c2_w, l0_b0_c2_b, l0_b0_c3_w, l0_b0_c3_b, l0_b0_down_w, l0_b0_down_b, l0_b1_c1_w, l0_b1_c1_b, l0_b1_c2_w, l0_b1_c2_b, l0_b1_c3_w, l0_b1_c3_b, l0_b2_c1_w, l0_b2_c1_b, l0_b2_c2_w, l0_b2_c2_b, l0_b2_c3_w, l0_b2_c3_b, l1_b0_c1_w, l1_b0_c1_b, l1_b0_c2_w, l1_b0_c2_b, l1_b0_c3_w, l1_b0_c3_b, l1_b0_down_w, l1_b0_down_b, l1_b1_c1_w, l1_b1_c1_b, l1_b1_c2_w, l1_b1_c2_b, l1_b1_c3_w, l1_b1_c3_b, l1_b2_c1_w, l1_b2_c1_b, l1_b2_c2_w, l1_b2_c2_b, l1_b2_c3_w, l1_b2_c3_b, l1_b3_c1_w, l1_b3_c1_b, l1_b3_c2_w, l1_b3_c2_b, l1_b3_c3_w, l1_b3_c3_b, l2_b0_c1_w, l2_b0_c1_b, l2_b0_c2_w, l2_b0_c2_b, l2_b0_c3_w, l2_b0_c3_b, l2_b0_down_w, l2_b0_down_b, l2_b1_c1_w, l2_b1_c1_b, l2_b1_c2_w, l2_b1_c2_b, l2_b1_c3_w, l2_b1_c3_b, l2_b2_c1_w, l2_b2_c1_b, l2_b2_c2_w, l2_b2_c2_b, l2_b2_c3_w, l2_b2_c3_b, l2_b3_c1_w, l2_b3_c1_b, l2_b3_c2_w, l2_b3_c2_b, l2_b3_c3_w, l2_b3_c3_b, l2_b4_c1_w, l2_b4_c1_b, l2_b4_c2_w, l2_b4_c2_b, l2_b4_c3_w, l2_b4_c3_b, l2_b5_c1_w, l2_b5_c1_b, l2_b5_c2_w, l2_b5_c2_b, l2_b5_c3_w, l2_b5_c3_b, l3_b0_c1_w, l3_b0_c1_b, l3_b0_c2_w, l3_b0_c2_b, l3_b0_c3_w, l3_b0_c3_b, l3_b0_down_w, l3_b0_down_b, l3_b1_c1_w, l3_b1_c1_b, l3_b1_c2_w, l3_b1_c2_b, l3_b1_c3_w, l3_b1_c3_b, l3_b2_c1_w, l3_b2_c1_b, l3_b2_c2_w, l3_b2_c2_b, l3_b2_c3_w, l3_b2_c3_b, embed_w, embed_b):
    d = dict(locals())
    images = d.pop('images')

    x = jnp.transpose(images, (0, 2, 3, 1)).astype(jnp.bfloat16)
    x = _conv_im2col(x, d['conv1_w'], d['conv1_b'], 7, 7, 2, 3, relu=True)
    x = _maxpool_3x3_s2(x)

    for li, (nblocks, first_stride) in enumerate(_LAYER_CFG):
        for bi in range(nblocks):
            s = first_stride if bi == 0 else 1
            p = "l%d_b%d_" % (li, bi)
            out = _conv1x1(x, d[p + 'c1_w'], d[p + 'c1_b'], relu=True)
            if s == 1:
                out = _conv3x3_s1(out, d[p + 'c2_w'], d[p + 'c2_b'])
            else:
                out = _conv_im2col(out, d[p + 'c2_w'], d[p + 'c2_b'],
                                   3, 3, s, 1, relu=True)
            if bi == 0:
                identity = _conv1x1(x, d[p + 'down_w'], d[p + 'down_b'],
                                    stride=s, relu=False)
            else:
                identity = x
            x = _conv1x1(out, d[p + 'c3_w'], d[p + 'c3_b'], relu=True,
                         residual=identity)

    return _avgpool_linear(x, d['embed_w'], d['embed_b'])
